# baseline (speedup 1.0000x reference)
import functools

import jax
import jax.numpy as jnp
from jax import lax
from jax.experimental import pallas as pl
from jax.experimental.pallas import tpu as pltpu
from jax.experimental.pallas import tpu_sc as plsc

_VOCAB = 1000000
_DIM = 64
_BATCH = 4096
_SEQ = 200
_ALPHA = 0.1
_PD = 128

_info = plsc.get_sparse_core_info()
_NC, _NS, _L = _info.num_cores, _info.num_subcores, _info.num_lanes
_NW = _NC * _NS
_BPW = _BATCH // _NW
_BG = _BPW // _L


def _sc_body(ids_hbm, tab_hbm, out_hbm, idx_v, g0, g1, g2, g3, o0, o1,
             sem_g0, sem_g1, sem_g2, sem_g3, sem_o0, sem_o1):
    wid = lax.axis_index("s") * _NC + lax.axis_index("c")
    b0 = wid * _BPW

    gbufs = (g0, g1, g2, g3)
    obufs = (o0, o1)
    sem_gs = (sem_g0, sem_g1, sem_g2, sem_g3)
    sem_os = (sem_o0, sem_o1)

    pltpu.sync_copy(ids_hbm.at[:, pl.ds(b0, _BPW)], idx_v)

    lanes = lax.iota(jnp.int32, _L)

    def gather(s, k):
        pltpu.make_async_copy(
            tab_hbm.at[idx_v.at[s]], gbufs[k], sem_gs[k]
        ).start()

    def wait_gather(s, k):
        pltpu.make_async_copy(
            tab_hbm.at[idx_v.at[s]], gbufs[k], sem_gs[k]
        ).wait()

    def store(s, j):
        pltpu.make_async_copy(
            obufs[j], out_hbm.at[s, :, pl.ds(b0, _BPW)], sem_os[j]
        ).start()

    def wait_store(s, j):
        pltpu.make_async_copy(
            obufs[j], out_hbm.at[s, :, pl.ds(b0, _BPW)], sem_os[j]
        ).wait()

    def compute(k, j, is_first):
        cur = gbufs[k]
        prv = gbufs[(k + 3) % 4]
        ob = obufs[j]

        def blk(i, _):
            dg = i >> 3
            h = i & 7
            ridx = lanes + h * _L
            for jj in range(_L):
                dd = (lanes + jj) & (_L - 1)
                cidx = dd + dg * _L
                cv = plsc.load_gather(cur, (ridx, cidx))
                if is_first:
                    ov = cv
                else:
                    pv = plsc.load_gather(prv, (ridx, cidx))
                    ov = cv * (1.0 + _ALPHA) - pv * _ALPHA
                plsc.store_scatter(ob, (cidx, ridx), ov)
            return 0

        lax.fori_loop(0, (_DIM // _L) * _BG, blk, 0)

    gather(0, 0)
    gather(1, 1)
    gather(2, 2)
    wait_gather(0, 0)
    compute(0, 0, True)
    store(0, 0)

    def step(i, _):
        for u in range(4):
            s = i * 4 + u - 3
            k = (u + 1) % 4
            j = (u + 1) % 2

            @pl.when(s < _SEQ)
            def _():
                @pl.when(s + 2 < _SEQ)
                def _():
                    gather(s + 2, (k + 2) % 4)

                wait_gather(s, k)

                @pl.when(s >= 2)
                def _():
                    wait_store(s - 2, j)

                store(s, j)
        return 0

    lax.fori_loop(1, (_SEQ + 2) // 4 + 1, step, 0)

    wait_store(_SEQ - 2, _SEQ % 2)
    wait_store(_SEQ - 1, (_SEQ - 1) % 2)


@jax.jit
def _gather_enhance(ids_t, tab_pad):
    mesh = plsc.VectorSubcoreMesh(core_axis_name="c", subcore_axis_name="s")
    run = functools.partial(
        pl.kernel,
        mesh=mesh,
        compiler_params=pltpu.CompilerParams(
            use_tc_tiling_on_sc=True, needs_layout_passes=False),
        out_type=jax.ShapeDtypeStruct((_SEQ, _DIM, _BATCH), jnp.float32),
        scratch_types=[
            pltpu.VMEM((_SEQ, _BPW), jnp.int32),
            pltpu.VMEM((_BPW, _PD), jnp.float32),
            pltpu.VMEM((_BPW, _PD), jnp.float32),
            pltpu.VMEM((_BPW, _PD), jnp.float32),
            pltpu.VMEM((_BPW, _PD), jnp.float32),
            pltpu.VMEM((_DIM, _BPW), jnp.float32),
            pltpu.VMEM((_DIM, _BPW), jnp.float32),
            pltpu.SemaphoreType.DMA,
            pltpu.SemaphoreType.DMA,
            pltpu.SemaphoreType.DMA,
            pltpu.SemaphoreType.DMA,
            pltpu.SemaphoreType.DMA,
            pltpu.SemaphoreType.DMA,
        ],
    )(_sc_body)
    return run(ids_t, tab_pad)


_FB = 512


def _fmt_body(tin, tout):
    tout[:, 0:_DIM] = lax.transpose(tin[...], (1, 0))


@jax.jit
def _tc_format(tab_t):
    grid = (_VOCAB + _FB - 1) // _FB
    return pl.pallas_call(
        _fmt_body,
        grid=(grid,),
        in_specs=[pl.BlockSpec((_DIM, _FB), lambda i: (0, i))],
        out_specs=pl.BlockSpec((_FB, _PD), lambda i: (i, 0)),
        out_shape=jax.ShapeDtypeStruct((_VOCAB, _PD), jnp.float32),
    )(tab_t)


def kernel(input_ids, table):
    ids_t = input_ids.T
    tab_pad = _tc_format(table.T)
    out_t = _gather_enhance(ids_t, tab_pad)
    return lax.transpose(out_t, (2, 0, 1))


# speedup vs baseline: 1.2835x; 1.2835x over previous
import functools

import jax
import jax.numpy as jnp
from jax import lax
from jax.experimental import pallas as pl
from jax.experimental.pallas import tpu as pltpu
from jax.experimental.pallas import tpu_sc as plsc

_VOCAB = 1000000
_DIM = 64
_BATCH = 4096
_SEQ = 200
_ALPHA = 0.1
_PD = 128
_G1 = 128
_G2 = _SEQ - _G1

_info = plsc.get_sparse_core_info()
_NC, _NS, _L = _info.num_cores, _info.num_subcores, _info.num_lanes
_NW = _NC * _NS
_BPW = _BATCH // _NW
_VPR = _DIM // _L


def _sc_body(ids_hbm, tab_hbm, out_hbm, idx_v, gb0, gb1, ob0, ob1,
             sem_g0, sem_g1, sem_o0, sem_o1):
    wid = lax.axis_index("s") * _NC + lax.axis_index("c")
    b0 = wid * _BPW

    gbufs = (gb0, gb1)
    obufs = (ob0, ob1)
    sem_gs = (sem_g0, sem_g1)
    sem_os = (sem_o0, sem_o1)

    pltpu.sync_copy(ids_hbm.at[pl.ds(b0 * _SEQ, _BPW * _SEQ)], idx_v)

    def gather(bi, k):
        pltpu.make_async_copy(
            tab_hbm.at[idx_v.at[pl.ds(bi * _SEQ, _G1)]],
            gbufs[k].at[pl.ds(0, _G1)], sem_gs[k]
        ).start()
        pltpu.make_async_copy(
            tab_hbm.at[idx_v.at[pl.ds(bi * _SEQ + _G1, _G2)]],
            gbufs[k].at[pl.ds(_G1, _G2)], sem_gs[k]
        ).start()

    def wait_gather(bi, k):
        pltpu.make_async_copy(
            tab_hbm.at[idx_v.at[pl.ds(bi * _SEQ, _G1)]],
            gbufs[k].at[pl.ds(0, _G1)], sem_gs[k]
        ).wait()
        pltpu.make_async_copy(
            tab_hbm.at[idx_v.at[pl.ds(bi * _SEQ + _G1, _G2)]],
            gbufs[k].at[pl.ds(_G1, _G2)], sem_gs[k]
        ).wait()

    def store(bi, k):
        pltpu.make_async_copy(
            obufs[k], out_hbm.at[b0 + bi], sem_os[k]
        ).start()

    def wait_store(bi, k):
        pltpu.make_async_copy(
            obufs[k], out_hbm.at[b0 + bi], sem_os[k]
        ).wait()

    def compute(k):
        gb, ob = gbufs[k], obufs[k]
        carry = []
        for q in range(_VPR):
            c = gb[0, pl.ds(q * _L, _L)]
            ob[0, pl.ds(q * _L, _L)] = c
            carry.append(c)

        def row(s, prev):
            cur = tuple(gb[s, pl.ds(q * _L, _L)] for q in range(_VPR))
            for q in range(_VPR):
                ob[s, pl.ds(q * _L, _L)] = (
                    cur[q] * (1.0 + _ALPHA) - prev[q] * _ALPHA)
            return cur

        lax.fori_loop(1, _SEQ, row, tuple(carry))

    gather(0, 0)

    def step(i, _):
        for u in range(2):
            bi = i * 2 + u

            @pl.when(bi + 1 < _BPW)
            def _():
                gather(bi + 1, 1 - u)

            wait_gather(bi, u)

            @pl.when(bi >= 2)
            def _():
                wait_store(bi - 2, u)

            compute(u)
            store(bi, u)
        return 0

    lax.fori_loop(0, _BPW // 2, step, 0)

    wait_store(_BPW - 2, 0)
    wait_store(_BPW - 1, 1)


@jax.jit
def _gather_enhance(ids, tab_pad):
    mesh = plsc.VectorSubcoreMesh(core_axis_name="c", subcore_axis_name="s")
    run = functools.partial(
        pl.kernel,
        mesh=mesh,
        compiler_params=pltpu.CompilerParams(
            use_tc_tiling_on_sc=True, needs_layout_passes=False),
        out_type=jax.ShapeDtypeStruct((_BATCH, _SEQ, _DIM), jnp.float32),
        scratch_types=[
            pltpu.VMEM((_BPW * _SEQ,), jnp.int32),
            pltpu.VMEM((_SEQ, _PD), jnp.float32),
            pltpu.VMEM((_SEQ, _PD), jnp.float32),
            pltpu.VMEM((_SEQ, _DIM), jnp.float32),
            pltpu.VMEM((_SEQ, _DIM), jnp.float32),
            pltpu.SemaphoreType.DMA,
            pltpu.SemaphoreType.DMA,
            pltpu.SemaphoreType.DMA,
            pltpu.SemaphoreType.DMA,
        ],
    )(_sc_body)
    return run(ids, tab_pad)


def kernel(input_ids, table):
    tab_pad = jnp.pad(table, ((0, 0), (0, _PD - _DIM)))
    return _gather_enhance(input_ids.reshape(_BATCH * _SEQ), tab_pad)
